# Initial kernel scaffold; baseline (speedup 1.0000x reference)
import jax
import jax.numpy as jnp
from jax.experimental import pallas as pl


def kernel(x, freqs_cis, Wq, Wk, Wv, Wo, Wg, W1, W2, W3, attn_norm_w, ffn_norm_w):
    raise NotImplementedError("write your pallas kernel here")


# speedup vs baseline: 1.5983x; 1.5983x over previous
import jax
import jax.numpy as jnp
from jax.experimental import pallas as pl
from jax.experimental.pallas import tpu as pltpu

S, D, H, KVH, DH, F, E = 2048, 1024, 16, 4, 64, 2048, 8
HALF = DH // 2
EPS = 1e-05
BQ = 1024
BS = 1024


def _rms(x, w):
    return x * jax.lax.rsqrt(jnp.mean(x * x, axis=-1, keepdims=True) + EPS) * w


def _qkv_kernel(x_ref, cos_ref, sin_ref, wn_ref, wq_ref, wk_ref, wv_ref,
                qa_ref, qb_ref, ka_ref, kb_ref, v_ref):
    x = x_ref[...]
    h = _rms(x, wn_ref[...]).astype(jnp.bfloat16)
    q = jnp.dot(h, wq_ref[...], preferred_element_type=jnp.float32)
    k = jnp.dot(h, wk_ref[...], preferred_element_type=jnp.float32)
    v = jnp.dot(h, wv_ref[...], preferred_element_type=jnp.float32)
    cq = jnp.tile(cos_ref[...], (1, H))
    sq = jnp.tile(sin_ref[...], (1, H))
    qa, qb = q[:, :H * HALF], q[:, H * HALF:]
    qa_ref[...] = (qa * cq - qb * sq).astype(jnp.bfloat16)
    qb_ref[...] = (qa * sq + qb * cq).astype(jnp.bfloat16)
    ck = jnp.tile(cos_ref[...], (1, KVH))
    sk = jnp.tile(sin_ref[...], (1, KVH))
    ka, kb = k[:, :KVH * HALF], k[:, KVH * HALF:]
    kra = (ka * ck - kb * sk).astype(jnp.bfloat16)
    krb = (ka * sk + kb * ck).astype(jnp.bfloat16)
    vb = v.astype(jnp.bfloat16)
    for g in range(KVH):
        ka_ref[g] = kra[:, g * HALF:(g + 1) * HALF]
        kb_ref[g] = krb[:, g * HALF:(g + 1) * HALF]
        v_ref[g] = vb[:, g * DH:(g + 1) * DH]


def _attn_kernel(qa_ref, qb_ref, ka_ref, kb_ref, v_ref, o_ref):
    scale = DH ** -0.5
    ka = ka_ref[0]
    kb = kb_ref[0]
    v = v_ref[0]
    dn = (((1,), (1,)), ((), ()))
    for h in range(H // KVH):
        qa = qa_ref[:, h * HALF:(h + 1) * HALF]
        qb = qb_ref[:, h * HALF:(h + 1) * HALF]
        s = jax.lax.dot_general(qa, ka, dn, preferred_element_type=jnp.float32)
        s += jax.lax.dot_general(qb, kb, dn, preferred_element_type=jnp.float32)
        e = jnp.exp(s * scale)
        denom = jnp.sum(e, axis=-1, keepdims=True)
        av = jnp.dot(e.astype(jnp.bfloat16), v, preferred_element_type=jnp.float32)
        o_ref[:, h * DH:(h + 1) * DH] = (av / denom).astype(jnp.bfloat16)


def _gate_kernel(o_ref, x_ref, wo_ref, wn_ref, wg_ref, h1_ref, t_ref, g_ref):
    proj = jnp.dot(o_ref[...], wo_ref[...], preferred_element_type=jnp.float32)
    h1 = x_ref[...] + proj
    h1_ref[...] = h1
    t = _rms(h1, wn_ref[...])
    t_ref[...] = t.astype(jnp.bfloat16)
    logits = jnp.dot(t, wg_ref[...], precision=jax.lax.Precision.HIGHEST,
                     preferred_element_type=jnp.float32)
    m = jnp.max(logits, axis=-1, keepdims=True)
    eg = jnp.exp(logits - m)
    g = eg / jnp.sum(eg, axis=-1, keepdims=True)
    lanes = jax.lax.broadcasted_iota(jnp.int32, (S, E), 1)
    m1 = jnp.max(g, axis=-1, keepdims=True)
    i1 = jnp.min(jnp.where(g >= m1, lanes, E), axis=-1, keepdims=True)
    oh1 = (lanes == i1)
    g2 = jnp.where(oh1, -1.0, g)
    m2 = jnp.max(g2, axis=-1, keepdims=True)
    i2 = jnp.min(jnp.where(g2 >= m2, lanes, E), axis=-1, keepdims=True)
    oh2 = (lanes == i2)
    wsum = m1 + m2
    g_ref[...] = jnp.where(oh1, m1 / wsum, jnp.where(oh2, m2 / wsum, 0.0))


def _moe_kernel(t_ref, g_ref, h1_ref, w1_ref, w2_ref, w3_ref, out_ref):
    e = pl.program_id(1)
    t = t_ref[...]
    a = jnp.dot(t, w1_ref[0], preferred_element_type=jnp.float32)
    b = jnp.dot(t, w3_ref[0], preferred_element_type=jnp.float32)
    hid = ((a / (1.0 + jnp.exp(-a))) * b).astype(jnp.bfloat16)
    eo = jnp.dot(hid, w2_ref[0], preferred_element_type=jnp.float32)
    lanes = jax.lax.broadcasted_iota(jnp.int32, (BS, E), 1)
    gcol = jnp.sum(jnp.where(lanes == e, g_ref[...], 0.0), axis=-1, keepdims=True)
    contrib = gcol * eo

    @pl.when(e == 0)
    def _():
        out_ref[...] = h1_ref[...] + contrib

    @pl.when(e != 0)
    def _():
        out_ref[...] += contrib


def kernel(x, freqs_cis, Wq, Wk, Wv, Wo, Wg, W1, W2, W3, attn_norm_w, ffn_norm_w):
    bf = jnp.bfloat16
    cos = jnp.cos(freqs_cis)
    sin = jnp.sin(freqs_cis)
    wq = Wq.reshape(D, H, HALF, 2)
    wqs = jnp.concatenate([wq[..., 0].reshape(D, H * HALF),
                           wq[..., 1].reshape(D, H * HALF)], axis=1).astype(bf)
    wk = Wk.reshape(D, KVH, HALF, 2)
    wks = jnp.concatenate([wk[..., 0].reshape(D, KVH * HALF),
                           wk[..., 1].reshape(D, KVH * HALF)], axis=1).astype(bf)

    qa, qb, ka, kb, v = pl.pallas_call(
        _qkv_kernel,
        out_shape=[
            jax.ShapeDtypeStruct((S, H * HALF), bf),
            jax.ShapeDtypeStruct((S, H * HALF), bf),
            jax.ShapeDtypeStruct((KVH, S, HALF), bf),
            jax.ShapeDtypeStruct((KVH, S, HALF), bf),
            jax.ShapeDtypeStruct((KVH, S, DH), bf),
        ],
    )(x, cos, sin, attn_norm_w.reshape(1, D), wqs, wks, Wv.astype(bf))

    ng = H // KVH
    o = pl.pallas_call(
        _attn_kernel,
        grid=(KVH, S // BQ),
        in_specs=[
            pl.BlockSpec((BQ, ng * HALF), lambda g, qb: (qb, g)),
            pl.BlockSpec((BQ, ng * HALF), lambda g, qb: (qb, g)),
            pl.BlockSpec((1, S, HALF), lambda g, qb: (g, 0, 0)),
            pl.BlockSpec((1, S, HALF), lambda g, qb: (g, 0, 0)),
            pl.BlockSpec((1, S, DH), lambda g, qb: (g, 0, 0)),
        ],
        out_specs=pl.BlockSpec((BQ, ng * DH), lambda g, qb: (qb, g)),
        out_shape=jax.ShapeDtypeStruct((S, H * DH), bf),
    )(qa, qb, ka, kb, v)

    h1, t, gate = pl.pallas_call(
        _gate_kernel,
        out_shape=[
            jax.ShapeDtypeStruct((S, D), jnp.float32),
            jax.ShapeDtypeStruct((S, D), bf),
            jax.ShapeDtypeStruct((S, E), jnp.float32),
        ],
    )(o, x, Wo.astype(bf), ffn_norm_w.reshape(1, D), Wg)

    out = pl.pallas_call(
        _moe_kernel,
        grid=(S // BS, E),
        in_specs=[
            pl.BlockSpec((BS, D), lambda sb, e: (sb, 0)),
            pl.BlockSpec((BS, E), lambda sb, e: (sb, 0)),
            pl.BlockSpec((BS, D), lambda sb, e: (sb, 0)),
            pl.BlockSpec((1, D, F), lambda sb, e: (e, 0, 0)),
            pl.BlockSpec((1, F, D), lambda sb, e: (e, 0, 0)),
            pl.BlockSpec((1, D, F), lambda sb, e: (e, 0, 0)),
        ],
        out_specs=pl.BlockSpec((BS, D), lambda sb, e: (sb, 0)),
        out_shape=jax.ShapeDtypeStruct((S, D), jnp.float32),
    )(t, gate, h1, W1.astype(bf), W2.astype(bf), W3.astype(bf))
    return out
